# Initial kernel scaffold; baseline (speedup 1.0000x reference)
import jax
import jax.numpy as jnp
from jax.experimental import pallas as pl


def kernel(x, Wr, br, W1, b1, W2, b2):
    raise NotImplementedError("write your pallas kernel here")


# speedup vs baseline: 2.9073x; 2.9073x over previous
import functools

import jax
import jax.numpy as jnp
from jax.experimental import pallas as pl
from jax.experimental.pallas import tpu as pltpu

S = 2048
D_MODEL = 1024
NUM_EXPERTS = 8
TOP_K = 2
D_FF = 4096
FFC = 512
NF = D_FF // FFC


def _router_body(x_ref, wr_ref, br_ref, w_ref, aux_ref):
    x = x_ref[...]
    logits = jax.lax.dot_general(
        x, wr_ref[...], (((1,), (0,)), ((), ())),
        preferred_element_type=jnp.float32) + br_ref[...]
    m = jnp.max(logits, axis=-1, keepdims=True)
    ex = jnp.exp(logits - m)
    probs = ex / jnp.sum(ex, axis=-1, keepdims=True)

    iota = jax.lax.broadcasted_iota(jnp.int32, probs.shape, 1)
    m1 = jnp.max(probs, axis=-1, keepdims=True)
    is1 = probs == m1
    idx1 = jnp.min(jnp.where(is1, iota, NUM_EXPERTS), axis=-1, keepdims=True)
    p2 = jnp.where(iota == idx1, -jnp.inf, probs)
    m2 = jnp.max(p2, axis=-1, keepdims=True)
    idx2 = jnp.min(jnp.where(p2 == m2, iota, NUM_EXPERTS), axis=-1,
                   keepdims=True)
    hw = (jnp.where(iota == idx1, m1, 0.0)
          + jnp.where(iota == idx2, m2, 0.0))
    hw = hw / jnp.maximum(m1 + m2, 1e-9)
    w_ref[...] = hw

    f_cnt = jnp.sum(jnp.where(iota == idx1, 1.0, 0.0), axis=0, keepdims=True)
    p_sum = jnp.sum(probs, axis=0, keepdims=True)
    aux = (NUM_EXPERTS / (S * S)) * jnp.sum(f_cnt * p_sum, keepdims=True)
    aux_ref[...] = aux.reshape(1, 1)


def _expert_body(x_ref, w1_ref, b1_ref, w2_ref, b2_ref, wt_ref, o_ref):
    e = pl.program_id(0)
    f = pl.program_id(1)

    @pl.when((e == 0) & (f == 0))
    def _init():
        o_ref[...] = jnp.zeros_like(o_ref)

    eio = jax.lax.broadcasted_iota(jnp.int32, (NUM_EXPERTS, 1), 0)
    onehot = jnp.where(eio == e, 1.0, 0.0)
    w_col = jax.lax.dot_general(
        wt_ref[...], onehot, (((1,), (0,)), ((), ())),
        preferred_element_type=jnp.float32)

    x = x_ref[...]
    h = jax.lax.dot_general(
        x, w1_ref[0], (((1,), (0,)), ((), ())),
        preferred_element_type=jnp.float32) + b1_ref[0]
    h = 0.5 * h * (1.0 + jax.lax.erf(h * (2.0 ** -0.5)))
    h = h * w_col
    contrib = jax.lax.dot_general(
        h, w2_ref[0], (((1,), (0,)), ((), ())),
        preferred_element_type=jnp.float32)
    o_ref[...] += contrib

    @pl.when(f == 0)
    def _bias():
        o_ref[...] += b2_ref[0] * w_col


@jax.jit
def kernel(x, Wr, br, W1, b1, W2, b2):
    B = x.shape[0]
    x2 = x.reshape(S, D_MODEL)

    w, aux = pl.pallas_call(
        _router_body,
        out_shape=[
            jax.ShapeDtypeStruct((S, NUM_EXPERTS), jnp.float32),
            jax.ShapeDtypeStruct((1, 1), jnp.float32),
        ],
    )(x2, Wr, br.reshape(1, NUM_EXPERTS))

    out = pl.pallas_call(
        _expert_body,
        grid=(NUM_EXPERTS, NF),
        in_specs=[
            pl.BlockSpec((S, D_MODEL), lambda e, f: (0, 0)),
            pl.BlockSpec((1, D_MODEL, FFC), lambda e, f: (e, 0, f)),
            pl.BlockSpec((1, 1, FFC), lambda e, f: (e, 0, f)),
            pl.BlockSpec((1, FFC, D_MODEL), lambda e, f: (e, f, 0)),
            pl.BlockSpec((1, 1, D_MODEL), lambda e, f: (e, 0, 0)),
            pl.BlockSpec((S, NUM_EXPERTS), lambda e, f: (0, 0)),
        ],
        out_specs=pl.BlockSpec((S, D_MODEL), lambda e, f: (0, 0)),
        out_shape=jax.ShapeDtypeStruct((S, D_MODEL), jnp.float32),
    )(x2, W1, b1.reshape(NUM_EXPERTS, 1, D_FF), W2,
      b2.reshape(NUM_EXPERTS, 1, D_MODEL), w)

    return out.reshape(B, S, D_MODEL), aux[0, 0]
